# baseline (speedup 1.0000x reference)
import jax
import jax.numpy as jnp
from jax.experimental import pallas as pl

_B, _H, _W, _C = 32, 16, 16, 64
_N = _H * _W
_R = _B * _N
_E = _N * (_N - 1)


def _template_kernel(tmpl_ref):
    k = jax.lax.broadcasted_iota(jnp.int32, (1, _E), 1)
    i = jnp.right_shift(k + jnp.right_shift(k, 8) + 1, 8)
    j = k - ((i << 8) - i)
    tmpl_ref[0:1] = i
    tmpl_ref[1:2] = j + (j >= i).astype(jnp.int32)


def _build_kernel(tmpl_ref, x_ref, edges_ref, nodes_ref, batch_ref):
    b = pl.program_id(0)
    edges_ref[...] = tmpl_ref[...] + b * _N

    rows = jax.lax.broadcasted_iota(jnp.int32, (_N, 1), 0)
    hr = jnp.right_shift(rows, 4).astype(jnp.float32) * (1.0 / (_H - 1))
    wc = jnp.bitwise_and(rows, _W - 1).astype(jnp.float32) * (1.0 / (_W - 1))
    nodes_ref[...] = jnp.concatenate([x_ref[...], hr, wc], axis=1)
    batch_ref[...] = jnp.full((_N, 1), b, dtype=jnp.int32)


def kernel(x, y):
    x2d = x.reshape(_R, _C)
    tmpl = pl.pallas_call(
        _template_kernel,
        out_shape=jax.ShapeDtypeStruct((2, _E), jnp.int32),
    )()
    edge_index, nodes, batch2 = pl.pallas_call(
        _build_kernel,
        grid=(_B,),
        in_specs=[
            pl.BlockSpec((2, _E), lambda b: (0, 0)),
            pl.BlockSpec((_N, _C), lambda b: (b, 0)),
        ],
        out_specs=[
            pl.BlockSpec((2, _E), lambda b: (0, b)),
            pl.BlockSpec((_N, _C + 2), lambda b: (b, 0)),
            pl.BlockSpec((_N, 1), lambda b: (b, 0)),
        ],
        out_shape=[
            jax.ShapeDtypeStruct((2, _B * _E), jnp.int32),
            jax.ShapeDtypeStruct((_R, _C + 2), jnp.float32),
            jax.ShapeDtypeStruct((_R, 1), jnp.int32),
        ],
    )(tmpl, x2d)
    batch_vec = batch2.reshape(_R)
    y_out = y.reshape(_B, -1)
    return nodes, edge_index, batch_vec, y_out
